# baseline (speedup 1.0000x reference)
import functools

import jax
import jax.numpy as jnp
from jax import lax
from jax.experimental import pallas as pl
from jax.experimental.pallas import tpu as pltpu
from jax.experimental.pallas import tpu_sc as plsc


def _mm_body(x_ref, w_ref, o_ref):
    o_ref[...] = jnp.dot(x_ref[...], w_ref[...],
                         preferred_element_type=jnp.float32)


def _tc_matmul(x, w, bm=2000):
    n, din = x.shape
    dout = w.shape[1]
    return pl.pallas_call(
        _mm_body,
        grid=(n // bm,),
        in_specs=[
            pl.BlockSpec((bm, din), lambda i: (i, 0)),
            pl.BlockSpec((din, dout), lambda i: (0, 0)),
        ],
        out_specs=pl.BlockSpec((bm, dout), lambda i: (i, 0)),
        out_shape=jax.ShapeDtypeStruct((n, dout), jnp.float32),
    )(x, w)


def _fused_body(p_ref, b_ref, w_ref, o_ref):
    t = p_ref[0] + p_ref[1] + b_ref[...]
    o_ref[...] = jnp.dot(jnp.maximum(t, 0.0), w_ref[...],
                         preferred_element_type=jnp.float32)


def _tc_fused(p, b, w, bm=2000):
    _, n, din = p.shape
    dout = w.shape[1]
    b2 = b.reshape(1, din)
    return pl.pallas_call(
        _fused_body,
        grid=(n // bm,),
        in_specs=[
            pl.BlockSpec((2, bm, din), lambda i: (0, i, 0)),
            pl.BlockSpec((1, din), lambda i: (0, 0)),
            pl.BlockSpec((din, dout), lambda i: (0, 0)),
        ],
        out_specs=pl.BlockSpec((bm, dout), lambda i: (i, 0)),
        out_shape=jax.ShapeDtypeStruct((n, dout), jnp.float32),
    )(p, b2, w)


def _final_body(p_ref, b_ref, fw_ref, fb_ref, emb_ref, logp_ref):
    emb = p_ref[0] + p_ref[1] + b_ref[...]
    emb_ref[...] = emb
    y = jnp.dot(emb, fw_ref[...], preferred_element_type=jnp.float32)
    y = y + fb_ref[...]
    ncls = logp_ref.shape[1]
    col = lax.broadcasted_iota(jnp.int32, y.shape, 1)
    valid = col < ncls
    ym = jnp.where(valid, y, -jnp.inf)
    m = jnp.max(ym, axis=1, keepdims=True)
    e = jnp.where(valid, jnp.exp(ym - m), 0.0)
    lse = jnp.log(jnp.sum(e, axis=1, keepdims=True))
    logp_ref[...] = (ym - m - lse)[:, :ncls]


def _tc_final(p, b, fcw, fcb, bm=2000):
    _, n, d = p.shape
    ncls = fcw.shape[1]
    pad = 128 - ncls
    fcw_p = jnp.pad(fcw, ((0, 0), (0, pad)))
    fcb_p = jnp.pad(fcb, ((0, pad),)).reshape(1, 128)
    b2 = b.reshape(1, d)
    return pl.pallas_call(
        _final_body,
        grid=(n // bm,),
        in_specs=[
            pl.BlockSpec((2, bm, d), lambda i: (0, i, 0)),
            pl.BlockSpec((1, d), lambda i: (0, 0)),
            pl.BlockSpec((d, 128), lambda i: (0, 0)),
            pl.BlockSpec((1, 128), lambda i: (0, 0)),
        ],
        out_specs=[
            pl.BlockSpec((bm, d), lambda i: (i, 0)),
            pl.BlockSpec((bm, ncls), lambda i: (i, 0)),
        ],
        out_shape=[
            jax.ShapeDtypeStruct((n, d), jnp.float32),
            jax.ShapeDtypeStruct((n, ncls), jnp.float32),
        ],
    )(p, b2, fcw_p, fcb_p)


_CH = 40
_NC = 2
_NS = 16
_RCH = 40
_NBUF = 4
_NPASS = 10


def _sc_edge_pass(h, src, dst, ew):
    n, d = h.shape
    e = src.shape[0]
    nw = _NC * _NS
    per_tile = e // nw
    per_pass = per_tile // _NPASS
    nchunk = per_pass // _CH
    nrch = n // _RCH
    nouter = (nchunk + _NBUF - 1) // _NBUF
    mesh = plsc.VectorSubcoreMesh(core_axis_name="c", subcore_axis_name="s")

    src4 = src.reshape(nw, _NPASS, nchunk, _CH)
    ew3 = ew.reshape(nw, _NPASS, per_pass)
    dst4 = dst.reshape(nw, _NPASS, nchunk, _CH)

    @functools.partial(
        pl.kernel,
        out_type=jax.ShapeDtypeStruct((_NC, n, d), jnp.float32),
        mesh=mesh,
        compiler_params=pltpu.CompilerParams(needs_layout_passes=False),
        scratch_types=[
            pltpu.VMEM_SHARED((n, d), jnp.float32),
            pltpu.VMEM((nchunk, _CH), jnp.int32),
            pltpu.VMEM((nchunk, _CH), jnp.int32),
            pltpu.VMEM((per_pass,), jnp.float32),
            [pltpu.VMEM((_CH, d), jnp.float32) for _ in range(_NBUF)],
            [pltpu.VMEM((_CH, d), jnp.float32) for _ in range(_NBUF)],
            [pltpu.SemaphoreType.DMA for _ in range(_NBUF)],
            [pltpu.SemaphoreType.DMA for _ in range(_NBUF)],
        ],
    )
    def edge_kernel(h_hbm, src_hbm, dst_hbm, ew_hbm, out_hbm,
                    acc, src_all, dst_all, ew_all, rows, scaled, gsem, ssem):
        c = lax.axis_index("c")
        s = lax.axis_index("s")
        w_id = c * _NS + s

        zvec = jnp.zeros((16,), jnp.float32)

        def _zrow(i, _):
            for k in range(d // 16):
                scaled[0][i, pl.ds(k * 16, 16)] = zvec
            return ()

        lax.fori_loop(0, _RCH, _zrow, ())

        ntrips = (nrch - s + _NS - 1) // _NS

        def _zcopy(t, _):
            row = (s + t * _NS) * _RCH
            pltpu.sync_copy(scaled[0], acc.at[pl.ds(row, _RCH)])
            return ()

        lax.fori_loop(0, ntrips, _zcopy, ())
        plsc.subcore_barrier()

        def _gather_start(j, b):
            return pltpu.async_copy(h_hbm.at[src_all.at[j]], rows[b], gsem[b])

        def _gather_wait(b):
            pltpu.make_async_copy(
                h_hbm.at[src_all.at[0]], rows[b], gsem[b]).wait()

        def _scatter_start(j, b):
            return pltpu.async_copy(scaled[b], acc.at[dst_all.at[j]],
                                    ssem[b], add=True)

        def _scatter_wait(b):
            pltpu.make_async_copy(scaled[b], acc.at[dst_all.at[0]],
                                  ssem[b]).wait()

        if _CH % 16 == 0:
            _groups = [(g * 16, 0) for g in range(_CH // 16)]
        else:
            _groups = [(g * 16, 0) for g in range(_CH // 16)]
            _groups.append((_CH - 16, 16 - _CH % 16))

        def _scale(j, b):
            for off, lo in _groups:
                wv = ew_all[pl.ds(j * _CH + off, 16)]
                for i in range(lo, 16):
                    wgt = wv[i]
                    row = off + i
                    for k in range(d // 16):
                        sl = pl.ds(k * 16, 16)
                        scaled[b][row, sl] = rows[b][row, sl] * wgt

        def _pass(p, _):
            pltpu.sync_copy(src_hbm.at[w_id, p], src_all)
            pltpu.sync_copy(dst_hbm.at[w_id, p], dst_all)
            pltpu.sync_copy(ew_hbm.at[w_id, p], ew_all)

            for b in range(_NBUF):
                _gather_start(b, b)

            def _outer(jo, _):
                for b in range(_NBUF):
                    j = jo * _NBUF + b

                    @pl.when(j < nchunk)
                    def _round():
                        _gather_wait(b)

                        @pl.when(j >= _NBUF)
                        def _():
                            _scatter_wait(b)

                        _scale(j, b)
                        _scatter_start(j, b)

                        @pl.when(j + _NBUF < nchunk)
                        def _():
                            _gather_start(j + _NBUF, b)
                return ()

            lax.fori_loop(0, nouter, _outer, ())

            for b in range(_NBUF):
                _scatter_wait(b)
            return ()

        lax.fori_loop(0, _NPASS, _pass, ())
        plsc.subcore_barrier()

        def _ocopy(t, _):
            row = (s + t * _NS) * _RCH
            pltpu.sync_copy(acc.at[pl.ds(row, _RCH)],
                            out_hbm.at[c, pl.ds(row, _RCH)])
            return ()

        lax.fori_loop(0, ntrips, _ocopy, ())

    return edge_kernel(h, src4, dst4, ew3)


def kernel(x, edge_index, edge_weight, W1, b1, W2, b2, W3, b3, fcW, fcb):
    src = edge_index[0].astype(jnp.int32)
    dst = edge_index[1].astype(jnp.int32)
    ew = edge_weight.astype(jnp.float32)

    h1 = _tc_matmul(x, W1)
    p1 = _sc_edge_pass(h1, src, dst, ew)
    h2 = _tc_fused(p1, b1, W2)
    p2 = _sc_edge_pass(h2, src, dst, ew)
    h3 = _tc_fused(p2, b2, W3)
    p3 = _sc_edge_pass(h3, src, dst, ew)
    emb, logp = _tc_final(p3, b3, fcW, fcb)
    return (emb, logp)


# speedup vs baseline: 1.0789x; 1.0789x over previous
import functools

import jax
import jax.numpy as jnp
from jax import lax
from jax.experimental import pallas as pl
from jax.experimental.pallas import tpu as pltpu
from jax.experimental.pallas import tpu_sc as plsc


def _mm_body(x_ref, w_ref, o_ref):
    o_ref[...] = jnp.dot(x_ref[...], w_ref[...],
                         preferred_element_type=jnp.float32)


def _tc_matmul(x, w, bm=2000):
    n, din = x.shape
    dout = w.shape[1]
    return pl.pallas_call(
        _mm_body,
        grid=(n // bm,),
        in_specs=[
            pl.BlockSpec((bm, din), lambda i: (i, 0)),
            pl.BlockSpec((din, dout), lambda i: (0, 0)),
        ],
        out_specs=pl.BlockSpec((bm, dout), lambda i: (i, 0)),
        out_shape=jax.ShapeDtypeStruct((n, dout), jnp.float32),
    )(x, w)


def _fused_body(p_ref, b_ref, w_ref, o_ref):
    t = p_ref[0] + p_ref[1] + b_ref[...]
    o_ref[...] = jnp.dot(jnp.maximum(t, 0.0), w_ref[...],
                         preferred_element_type=jnp.float32)


def _tc_fused(p, b, w, bm=2000):
    _, n, din = p.shape
    dout = w.shape[1]
    b2 = b.reshape(1, din)
    return pl.pallas_call(
        _fused_body,
        grid=(n // bm,),
        in_specs=[
            pl.BlockSpec((2, bm, din), lambda i: (0, i, 0)),
            pl.BlockSpec((1, din), lambda i: (0, 0)),
            pl.BlockSpec((din, dout), lambda i: (0, 0)),
        ],
        out_specs=pl.BlockSpec((bm, dout), lambda i: (i, 0)),
        out_shape=jax.ShapeDtypeStruct((n, dout), jnp.float32),
    )(p, b2, w)


def _final_body(p_ref, b_ref, fw_ref, fb_ref, emb_ref, logp_ref):
    emb = p_ref[0] + p_ref[1] + b_ref[...]
    emb_ref[...] = emb
    y = jnp.dot(emb, fw_ref[...], preferred_element_type=jnp.float32)
    y = y + fb_ref[...]
    ncls = logp_ref.shape[1]
    col = lax.broadcasted_iota(jnp.int32, y.shape, 1)
    valid = col < ncls
    ym = jnp.where(valid, y, -jnp.inf)
    m = jnp.max(ym, axis=1, keepdims=True)
    e = jnp.where(valid, jnp.exp(ym - m), 0.0)
    lse = jnp.log(jnp.sum(e, axis=1, keepdims=True))
    logp_ref[...] = (ym - m - lse)[:, :ncls]


def _tc_final(p, b, fcw, fcb, bm=2000):
    _, n, d = p.shape
    ncls = fcw.shape[1]
    pad = 128 - ncls
    fcw_p = jnp.pad(fcw, ((0, 0), (0, pad)))
    fcb_p = jnp.pad(fcb, ((0, pad),)).reshape(1, 128)
    b2 = b.reshape(1, d)
    return pl.pallas_call(
        _final_body,
        grid=(n // bm,),
        in_specs=[
            pl.BlockSpec((2, bm, d), lambda i: (0, i, 0)),
            pl.BlockSpec((1, d), lambda i: (0, 0)),
            pl.BlockSpec((d, 128), lambda i: (0, 0)),
            pl.BlockSpec((1, 128), lambda i: (0, 0)),
        ],
        out_specs=[
            pl.BlockSpec((bm, d), lambda i: (i, 0)),
            pl.BlockSpec((bm, ncls), lambda i: (i, 0)),
        ],
        out_shape=[
            jax.ShapeDtypeStruct((n, d), jnp.float32),
            jax.ShapeDtypeStruct((n, ncls), jnp.float32),
        ],
    )(p, b2, fcw_p, fcb_p)


_CH = 40
_NC = 2
_NS = 16
_RCH = 40
_NG = 4
_NSB = 2
_NPASS = 5


def _sc_edge_pass(h, src, dst, ew):
    n, d = h.shape
    e = src.shape[0]
    nw = _NC * _NS
    per_tile = e // nw
    per_pass = per_tile // _NPASS
    nchunk = per_pass // _CH
    nrch = n // _RCH
    nouter = (nchunk + _NG - 1) // _NG
    mesh = plsc.VectorSubcoreMesh(core_axis_name="c", subcore_axis_name="s")

    src4 = src.reshape(nw, _NPASS, nchunk, _CH)
    ew3 = ew.reshape(nw, _NPASS, per_pass)
    dst4 = dst.reshape(nw, _NPASS, nchunk, _CH)

    @functools.partial(
        pl.kernel,
        out_type=jax.ShapeDtypeStruct((_NC, n, d), jnp.float32),
        mesh=mesh,
        compiler_params=pltpu.CompilerParams(needs_layout_passes=False),
        scratch_types=[
            pltpu.VMEM_SHARED((n, d), jnp.float32),
            pltpu.VMEM((nchunk, _CH), jnp.int32),
            pltpu.VMEM((nchunk, _CH), jnp.int32),
            pltpu.VMEM((per_pass,), jnp.float32),
            [pltpu.VMEM((_CH, d), jnp.float32) for _ in range(_NG)],
            [pltpu.VMEM((_CH, d), jnp.float32) for _ in range(_NSB)],
            [pltpu.SemaphoreType.DMA for _ in range(_NG)],
            [pltpu.SemaphoreType.DMA for _ in range(_NSB)],
        ],
    )
    def edge_kernel(h_hbm, src_hbm, dst_hbm, ew_hbm, out_hbm,
                    acc, src_all, dst_all, ew_all, rows, scaled, gsem, ssem):
        c = lax.axis_index("c")
        s = lax.axis_index("s")
        w_id = c * _NS + s

        zvec = jnp.zeros((16,), jnp.float32)

        def _zrow(i, _):
            for k in range(d // 16):
                scaled[0][i, pl.ds(k * 16, 16)] = zvec
            return ()

        lax.fori_loop(0, _RCH, _zrow, ())

        ntrips = (nrch - s + _NS - 1) // _NS

        def _zcopy(t, _):
            row = (s + t * _NS) * _RCH
            pltpu.sync_copy(scaled[0], acc.at[pl.ds(row, _RCH)])
            return ()

        lax.fori_loop(0, ntrips, _zcopy, ())
        plsc.subcore_barrier()

        def _gather_start(j, b):
            return pltpu.async_copy(h_hbm.at[src_all.at[j]], rows[b], gsem[b])

        def _gather_wait(b):
            pltpu.make_async_copy(
                h_hbm.at[src_all.at[0]], rows[b], gsem[b]).wait()

        def _scatter_start(j, b):
            return pltpu.async_copy(scaled[b], acc.at[dst_all.at[j]],
                                    ssem[b], add=True)

        def _scatter_wait(b):
            pltpu.make_async_copy(scaled[b], acc.at[dst_all.at[0]],
                                  ssem[b]).wait()

        if _CH % 16 == 0:
            _groups = [(g * 16, 0) for g in range(_CH // 16)]
        else:
            _groups = [(g * 16, 0) for g in range(_CH // 16)]
            _groups.append((_CH - 16, 16 - _CH % 16))

        def _scale(j, b, bs):
            for off, lo in _groups:
                wv = ew_all[pl.ds(j * _CH + off, 16)]
                for i in range(lo, 16):
                    wgt = wv[i]
                    row = off + i
                    for k in range(d // 16):
                        sl = pl.ds(k * 16, 16)
                        scaled[bs][row, sl] = rows[b][row, sl] * wgt

        def _pass(p, _):
            pltpu.sync_copy(src_hbm.at[w_id, p], src_all)
            pltpu.sync_copy(dst_hbm.at[w_id, p], dst_all)
            pltpu.sync_copy(ew_hbm.at[w_id, p], ew_all)

            for b in range(_NG):
                _gather_start(b, b)

            def _outer(jo, _):
                for b in range(_NG):
                    j = jo * _NG + b
                    bs = b % _NSB

                    @pl.when(j < nchunk)
                    def _round():
                        _gather_wait(b)

                        @pl.when(j >= _NSB)
                        def _():
                            _scatter_wait(bs)

                        _scale(j, b, bs)
                        _scatter_start(j, bs)

                        @pl.when(j + _NG < nchunk)
                        def _():
                            _gather_start(j + _NG, b)
                return ()

            lax.fori_loop(0, nouter, _outer, ())

            for b in range(_NSB):
                _scatter_wait(b)
            return ()

        lax.fori_loop(0, _NPASS, _pass, ())
        plsc.subcore_barrier()

        def _ocopy(t, _):
            row = (s + t * _NS) * _RCH
            pltpu.sync_copy(acc.at[pl.ds(row, _RCH)],
                            out_hbm.at[c, pl.ds(row, _RCH)])
            return ()

        lax.fori_loop(0, ntrips, _ocopy, ())

    return edge_kernel(h, src4, dst4, ew3)


def kernel(x, edge_index, edge_weight, W1, b1, W2, b2, W3, b3, fcW, fcb):
    src = edge_index[0].astype(jnp.int32)
    dst = edge_index[1].astype(jnp.int32)
    ew = edge_weight.astype(jnp.float32)

    h1 = _tc_matmul(x, W1)
    p1 = _sc_edge_pass(h1, src, dst, ew)
    h2 = _tc_fused(p1, b1, W2)
    p2 = _sc_edge_pass(h2, src, dst, ew)
    h3 = _tc_fused(p2, b2, W3)
    p3 = _sc_edge_pass(h3, src, dst, ew)
    emb, logp = _tc_final(p3, b3, fcW, fcb)
    return (emb, logp)
